# Initial kernel scaffold; baseline (speedup 1.0000x reference)
import functools

import jax
import jax.numpy as jnp
from jax import lax
from jax.experimental import pallas as pl
from jax.experimental.pallas import tpu as pltpu
from jax.experimental.pallas import tpu_sc as plsc

N = 50000
E = 800000
D_IN = 8
D_HID = 128

NC = 2
NS = 16
NW = NC * NS

WIN = 128
NB = 8
WINDOWS_PER_TILE = 200
E_PAD = NW * WINDOWS_PER_TILE * WIN
TRASH = N
ACC_ROWS = N + 8
ROWS_PER_TILE = N // NS
ZROWS = 625


def _sc_agg(table, src2d, dst2d, w, n_chunks):
    mesh = plsc.VectorSubcoreMesh(core_axis_name="c", subcore_axis_name="s")
    groups = WINDOWS_PER_TILE // NB

    @functools.partial(
        pl.kernel,
        mesh=mesh,
        out_type=jax.ShapeDtypeStruct((NC, n_chunks, N, w), jnp.float32),
        scratch_types=[
            pltpu.VMEM((NB, WIN), jnp.int32),
            pltpu.VMEM((NB, WIN), jnp.int32),
            pltpu.VMEM((NB, WIN, w), jnp.float32),
            pltpu.VMEM((ZROWS, w), jnp.float32),
            pltpu.VMEM_SHARED((ACC_ROWS, w), jnp.float32),
            pltpu.SemaphoreType.DMA,
        ],
    )
    def k(tbl_hbm, src_hbm, dst_hbm, p_hbm, srcb, dstb, rows, zbuf, accum, sem):
        core = lax.axis_index("c")
        sub = lax.axis_index("s")
        tile = sub * NC + core
        tile_row0 = tile * WINDOWS_PER_TILE

        @pl.loop(0, ZROWS)
        def _(i):
            for j in range(w // 16):
                zbuf[i, pl.ds(j * 16, 16)] = jnp.zeros((16,), jnp.float32)

        for chunk in range(n_chunks):
            tbl = tbl_hbm if n_chunks == 1 else tbl_hbm.at[chunk]

            for z in range(ROWS_PER_TILE // ZROWS):
                pltpu.sync_copy(
                    zbuf, accum.at[pl.ds(sub * ROWS_PER_TILE + z * ZROWS, ZROWS)]
                )
            plsc.subcore_barrier()

            @pl.loop(0, groups)
            def _(g):
                row0 = tile_row0 + g * NB
                pltpu.sync_copy(src_hbm.at[pl.ds(row0, NB)], srcb)
                pltpu.sync_copy(dst_hbm.at[pl.ds(row0, NB)], dstb)
                cps = [
                    pltpu.async_copy(tbl.at[srcb.at[j]], rows.at[j], sem)
                    for j in range(NB)
                ]
                for cp in cps:
                    cp.wait()
                for j in range(NB):
                    pltpu.sync_copy(rows.at[j], accum.at[dstb.at[j]], add=True)

            plsc.subcore_barrier()

            for z in range(ROWS_PER_TILE // ZROWS):
                r0 = sub * ROWS_PER_TILE + z * ZROWS
                pltpu.sync_copy(
                    accum.at[pl.ds(r0, ZROWS)],
                    p_hbm.at[core, chunk].at[pl.ds(r0, ZROWS)],
                )
            plsc.subcore_barrier()

    return k(table, src2d, dst2d)


BN = 2000


def _tc_layer1(p1, x, wl, wr, b):
    def body(p_ref, x_ref, wl_ref, wr_ref, b_ref, h_ref, dinv_ref):
        p = p_ref[0] + p_ref[1]
        dinv = 1.0 / jnp.maximum(p[:, 8:9], 1.0)
        agg = p[:, :D_IN] * dinv
        h = (
            jnp.dot(agg, wl_ref[...], preferred_element_type=jnp.float32)
            + jnp.dot(x_ref[...], wr_ref[...], preferred_element_type=jnp.float32)
            + b_ref[...]
        )
        h = jnp.maximum(h, 0.0)
        for c in range(4):
            h_ref[c] = h[:, c * 32 : (c + 1) * 32]
        dinv_ref[...] = dinv

    return pl.pallas_call(
        body,
        grid=(N // BN,),
        in_specs=[
            pl.BlockSpec((NC, BN, 16), lambda i: (0, i, 0)),
            pl.BlockSpec((BN, D_IN), lambda i: (i, 0)),
            pl.BlockSpec((D_IN, D_HID), lambda i: (0, 0)),
            pl.BlockSpec((D_IN, D_HID), lambda i: (0, 0)),
            pl.BlockSpec((1, D_HID), lambda i: (0, 0)),
        ],
        out_specs=[
            pl.BlockSpec((4, BN, 32), lambda i: (0, i, 0)),
            pl.BlockSpec((BN, 1), lambda i: (i, 0)),
        ],
        out_shape=[
            jax.ShapeDtypeStruct((4, N, 32), jnp.float32),
            jax.ShapeDtypeStruct((N, 1), jnp.float32),
        ],
    )(p1, x, wl, wr, b)


def _tc_layer(p, hprev, dinv, wl, wr, b, relu, chunked_out):
    def body(p_ref, h_ref, d_ref, wl_ref, wr_ref, b_ref, o_ref):
        dinv = d_ref[...]
        agg = jnp.concatenate(
            [(p_ref[0, c] + p_ref[1, c]) * dinv for c in range(4)], axis=1
        )
        h = jnp.concatenate([h_ref[c] for c in range(4)], axis=1)
        out = (
            jnp.dot(agg, wl_ref[...], preferred_element_type=jnp.float32)
            + jnp.dot(h, wr_ref[...], preferred_element_type=jnp.float32)
            + b_ref[...]
            + h
        )
        if relu:
            out = jnp.maximum(out, 0.0)
        if chunked_out:
            for c in range(4):
                o_ref[c] = out[:, c * 32 : (c + 1) * 32]
        else:
            o_ref[...] = out

    if chunked_out:
        out_spec = pl.BlockSpec((4, BN, 32), lambda i: (0, i, 0))
        out_shape = jax.ShapeDtypeStruct((4, N, 32), jnp.float32)
    else:
        out_spec = pl.BlockSpec((BN, D_HID), lambda i: (i, 0))
        out_shape = jax.ShapeDtypeStruct((N, D_HID), jnp.float32)

    return pl.pallas_call(
        body,
        grid=(N // BN,),
        in_specs=[
            pl.BlockSpec((NC, 4, BN, 32), lambda i: (0, 0, i, 0)),
            pl.BlockSpec((4, BN, 32), lambda i: (0, i, 0)),
            pl.BlockSpec((BN, 1), lambda i: (i, 0)),
            pl.BlockSpec((D_HID, D_HID), lambda i: (0, 0)),
            pl.BlockSpec((D_HID, D_HID), lambda i: (0, 0)),
            pl.BlockSpec((1, D_HID), lambda i: (0, 0)),
        ],
        out_specs=[out_spec],
        out_shape=[out_shape],
    )(p, hprev, dinv, wl, wr, b)[0]


def kernel(x, edge_index, Wl1, Wr1, b1, Wl2, Wr2, b2, Wl3, Wr3, b3):
    src = edge_index[0].astype(jnp.int32)
    dst = edge_index[1].astype(jnp.int32)
    pad = E_PAD - E
    src_p = jnp.concatenate([src, jnp.zeros((pad,), jnp.int32)])
    dst_p = jnp.concatenate([dst, jnp.full((pad,), TRASH, jnp.int32)])
    src2d = src_p.reshape(E_PAD // WIN, WIN)
    dst2d = dst_p.reshape(E_PAD // WIN, WIN)

    xpad = jnp.concatenate(
        [x, jnp.ones((N, 1), jnp.float32), jnp.zeros((N, 7), jnp.float32)], axis=1
    )

    b1r = b1.reshape(1, D_HID)
    b2r = b2.reshape(1, D_HID)
    b3r = b3.reshape(1, D_HID)

    p1 = _sc_agg(xpad, src2d, dst2d, 16, 1)
    h1, dinv = _tc_layer1(p1.reshape(NC, N, 16), x, Wl1, Wr1, b1r)

    p2 = _sc_agg(h1, src2d, dst2d, 32, 4)
    h2 = _tc_layer(p2, h1, dinv, Wl2, Wr2, b2r, relu=True, chunked_out=True)

    p3 = _sc_agg(h2, src2d, dst2d, 32, 4)
    out = _tc_layer(p3, h2, dinv, Wl3, Wr3, b3r, relu=False, chunked_out=False)
    return out


# speedup vs baseline: 3.0458x; 3.0458x over previous
import functools

import jax
import jax.numpy as jnp
from jax import lax
from jax.experimental import pallas as pl
from jax.experimental.pallas import tpu as pltpu
from jax.experimental.pallas import tpu_sc as plsc

N = 50000
E = 800000
D_IN = 8
D_HID = 128

NC = 2
NS = 16
NW = NC * NS

WIN = 128
WINDOWS_PER_TILE = 200
E_PAD = NW * WINDOWS_PER_TILE * WIN
TRASH = N
N_PAD = 51200
ROWS_PER_TILE = N_PAD // NS
ZROWS = 800


def _sc_agg(table, src2d, dst2d, zeros, w, n_chunks, nb):
    mesh = plsc.VectorSubcoreMesh(core_axis_name="c", subcore_axis_name="s")
    groups = WINDOWS_PER_TILE // nb

    @functools.partial(
        pl.kernel,
        mesh=mesh,
        out_type=jax.ShapeDtypeStruct((NC, n_chunks, N_PAD, w), jnp.float32),
        scratch_types=[
            pltpu.VMEM((nb, WIN), jnp.int32),
            pltpu.VMEM((nb, WIN), jnp.int32),
            pltpu.VMEM((nb, WIN, w), jnp.float32),
            pltpu.VMEM_SHARED((N_PAD, w), jnp.float32),
            pltpu.SemaphoreType.DMA,
        ],
        compiler_params=pltpu.CompilerParams(use_tc_tiling_on_sc=False),
    )
    def k(tbl_hbm, src_hbm, dst_hbm, z_hbm, p_hbm, srcb, dstb, rows, accum, sem):
        core = lax.axis_index("c")
        sub = lax.axis_index("s")
        tile = sub * NC + core
        tile_row0 = tile * WINDOWS_PER_TILE

        for chunk in range(n_chunks):
            tbl = tbl_hbm if n_chunks == 1 else tbl_hbm.at[chunk]

            zcps = [
                pltpu.async_copy(
                    z_hbm, accum.at[pl.ds(sub * ROWS_PER_TILE + z * ZROWS, ZROWS)], sem
                )
                for z in range(ROWS_PER_TILE // ZROWS)
            ]
            for cp in zcps:
                cp.wait()
            plsc.subcore_barrier()

            @pl.loop(0, groups)
            def _(g):
                row0 = tile_row0 + g * nb
                pltpu.sync_copy(src_hbm.at[pl.ds(row0, nb)], srcb)
                pltpu.sync_copy(dst_hbm.at[pl.ds(row0, nb)], dstb)
                cps = [
                    pltpu.async_copy(tbl.at[srcb.at[j]], rows.at[j], sem)
                    for j in range(nb)
                ]
                for cp in cps:
                    cp.wait()
                for j in range(nb):
                    pltpu.sync_copy(rows.at[j], accum.at[dstb.at[j]], add=True)

            plsc.subcore_barrier()

            dcps = [
                pltpu.async_copy(
                    accum.at[pl.ds(sub * ROWS_PER_TILE + z * ZROWS, ZROWS)],
                    p_hbm.at[core, chunk].at[pl.ds(sub * ROWS_PER_TILE + z * ZROWS, ZROWS)],
                    sem,
                )
                for z in range(ROWS_PER_TILE // ZROWS)
            ]
            for cp in dcps:
                cp.wait()
            plsc.subcore_barrier()

    return k(table, src2d, dst2d, zeros)


BN = 2000


def _tc_layer1(p1, x, wl, wr, b):
    def body(p_ref, x_ref, wl_ref, wr_ref, b_ref, h_ref, dinv_ref):
        p = p_ref[0] + p_ref[1]
        dinv = 1.0 / jnp.maximum(p[:, 8:9], 1.0)
        agg = p[:, :D_IN] * dinv
        h = (
            jnp.dot(agg, wl_ref[...], preferred_element_type=jnp.float32)
            + jnp.dot(x_ref[...], wr_ref[...], preferred_element_type=jnp.float32)
            + b_ref[...]
        )
        h = jnp.maximum(h, 0.0)
        for c in range(4):
            h_ref[c] = h[:, c * 32 : (c + 1) * 32]
        dinv_ref[...] = dinv

    return pl.pallas_call(
        body,
        grid=(N // BN,),
        in_specs=[
            pl.BlockSpec((NC, BN, 16), lambda i: (0, i, 0)),
            pl.BlockSpec((BN, D_IN), lambda i: (i, 0)),
            pl.BlockSpec((D_IN, D_HID), lambda i: (0, 0)),
            pl.BlockSpec((D_IN, D_HID), lambda i: (0, 0)),
            pl.BlockSpec((1, D_HID), lambda i: (0, 0)),
        ],
        out_specs=[
            pl.BlockSpec((4, BN, 32), lambda i: (0, i, 0)),
            pl.BlockSpec((BN, 1), lambda i: (i, 0)),
        ],
        out_shape=[
            jax.ShapeDtypeStruct((4, N, 32), jnp.float32),
            jax.ShapeDtypeStruct((N, 1), jnp.float32),
        ],
    )(p1, x, wl, wr, b)


def _tc_layer(p, hprev, dinv, wl, wr, b, relu, chunked_out):
    def body(p_ref, h_ref, d_ref, wl_ref, wr_ref, b_ref, o_ref):
        dinv = d_ref[...]
        agg = jnp.concatenate(
            [(p_ref[0, c] + p_ref[1, c]) * dinv for c in range(4)], axis=1
        )
        h = jnp.concatenate([h_ref[c] for c in range(4)], axis=1)
        out = (
            jnp.dot(agg, wl_ref[...], preferred_element_type=jnp.float32)
            + jnp.dot(h, wr_ref[...], preferred_element_type=jnp.float32)
            + b_ref[...]
            + h
        )
        if relu:
            out = jnp.maximum(out, 0.0)
        if chunked_out:
            for c in range(4):
                o_ref[c] = out[:, c * 32 : (c + 1) * 32]
        else:
            o_ref[...] = out

    if chunked_out:
        out_spec = pl.BlockSpec((4, BN, 32), lambda i: (0, i, 0))
        out_shape = jax.ShapeDtypeStruct((4, N, 32), jnp.float32)
    else:
        out_spec = pl.BlockSpec((BN, D_HID), lambda i: (i, 0))
        out_shape = jax.ShapeDtypeStruct((N, D_HID), jnp.float32)

    return pl.pallas_call(
        body,
        grid=(N // BN,),
        in_specs=[
            pl.BlockSpec((NC, 4, BN, 32), lambda i: (0, 0, i, 0)),
            pl.BlockSpec((4, BN, 32), lambda i: (0, i, 0)),
            pl.BlockSpec((BN, 1), lambda i: (i, 0)),
            pl.BlockSpec((D_HID, D_HID), lambda i: (0, 0)),
            pl.BlockSpec((D_HID, D_HID), lambda i: (0, 0)),
            pl.BlockSpec((1, D_HID), lambda i: (0, 0)),
        ],
        out_specs=[out_spec],
        out_shape=[out_shape],
    )(p, hprev, dinv, wl, wr, b)[0]


def kernel(x, edge_index, Wl1, Wr1, b1, Wl2, Wr2, b2, Wl3, Wr3, b3):
    src = edge_index[0].astype(jnp.int32)
    dst = edge_index[1].astype(jnp.int32)
    pad = E_PAD - E
    src_p = jnp.concatenate([src, jnp.zeros((pad,), jnp.int32)])
    dst_p = jnp.concatenate([dst, jnp.full((pad,), TRASH, jnp.int32)])
    src2d = src_p.reshape(E_PAD // WIN, WIN)
    dst2d = dst_p.reshape(E_PAD // WIN, WIN)

    xpad = jnp.concatenate(
        [x, jnp.ones((N, 1), jnp.float32), jnp.zeros((N, 7), jnp.float32)], axis=1
    )

    b1r = b1.reshape(1, D_HID)
    b2r = b2.reshape(1, D_HID)
    b3r = b3.reshape(1, D_HID)

    z16 = jnp.zeros((ZROWS, 16), jnp.float32)
    z32 = jnp.zeros((ZROWS, 32), jnp.float32)

    p1 = _sc_agg(xpad, src2d, dst2d, z16, 16, 1, 8)
    h1, dinv = _tc_layer1(p1.reshape(NC, N_PAD, 16), x, Wl1, Wr1, b1r)

    p2 = _sc_agg(h1, src2d, dst2d, z32, 32, 4, 4)
    h2 = _tc_layer(p2, h1, dinv, Wl2, Wr2, b2r, relu=True, chunked_out=True)

    p3 = _sc_agg(h2, src2d, dst2d, z32, 32, 4, 4)
    out = _tc_layer(p3, h2, dinv, Wl3, Wr3, b3r, relu=False, chunked_out=False)
    return out


# speedup vs baseline: 5.5243x; 1.8138x over previous
import dataclasses
import functools

import jax
import jax.numpy as jnp
from jax import lax
from jax.experimental import pallas as pl
from jax.experimental.pallas import tpu as pltpu
from jax.experimental.pallas import tpu_sc as plsc

N = 50000
E = 800000
D_IN = 8
D_HID = 128

NC = 2
NS = 16
NW = NC * NS

WIN = 128
NBF = 4
RUN_PAD = NBF * WIN
EPT = 25600
E_PAD = NW * EPT
WROWS = EPT // WIN

NBUCK = 8
BH = 6400
ACC_TRASH = 128
ACC_ROWS = BH + ACC_TRASH
N_OUT = NBUCK * BH
STG_CH = (EPT + NBUCK * (RUN_PAD - 1) + WIN - 1) // WIN

_MESH = plsc.VectorSubcoreMesh(core_axis_name="c", subcore_axis_name="s")

_SC_PARAMS = pltpu.CompilerParams(use_tc_tiling_on_sc=False)
if "needs_layout_passes" in pltpu.CompilerParams.__dataclass_fields__:
    _SC_PARAMS = dataclasses.replace(_SC_PARAMS, needs_layout_passes=False)


def _cnt_padded(cnt):
    return ((cnt + RUN_PAD - 1) // RUN_PAD) * RUN_PAD


def _sc_sort(src2d, dst2d):
    @functools.partial(
        pl.kernel,
        mesh=_MESH,
        out_type=[
            jax.ShapeDtypeStruct((NW, STG_CH, WIN), jnp.int32),
            jax.ShapeDtypeStruct((NW, STG_CH, WIN), jnp.int32),
            jax.ShapeDtypeStruct((NW, 16), jnp.int32),
        ],
        scratch_types=[
            pltpu.VMEM((8, WIN), jnp.int32),
            pltpu.VMEM((8, WIN), jnp.int32),
            pltpu.VMEM((STG_CH, WIN), jnp.int32),
            pltpu.VMEM((STG_CH, WIN), jnp.int32),
            pltpu.VMEM((1, 16), jnp.int32),
            pltpu.SemaphoreType.DMA,
        ],
        compiler_params=_SC_PARAMS,
    )
    def k(src_hbm, dst_hbm, rs_hbm, rd_hbm, h_hbm, srcw, dstw, stgs, stgd,
          histv, sem):
        core = lax.axis_index("c")
        sub = lax.axis_index("s")
        tile = sub * NC + core
        trow0 = tile * WROWS
        lanes = lax.iota(jnp.int32, 16)

        def p1_win(wi, cnts):
            pltpu.sync_copy(dst_hbm.at[pl.ds(trow0 + wi * 8, 8)], dstw)
            new = list(cnts)
            for r in range(8):
                for ci in range(8):
                    dv = dstw[r, pl.ds(ci * 16, 16)]
                    bv = dv // BH
                    for b in range(NBUCK):
                        ind = jnp.where(bv == b, 1, 0)
                        new[b] = new[b] + jnp.sum(ind)
            return tuple(new)

        cnts = lax.fori_loop(0, WROWS // 8, p1_win,
                             tuple(jnp.int32(0) for _ in range(NBUCK)))

        offs = []
        acc = jnp.int32(0)
        for b in range(NBUCK):
            offs.append(acc)
            acc = acc + _cnt_padded(cnts[b])

        def prefill(i, _):
            for c in range(8):
                base = i * 128 + c * 16
                stgs[i, pl.ds(c * 16, 16)] = (base + lanes * 137) % N
                stgd[i, pl.ds(c * 16, 16)] = BH + (base + lanes) % ACC_TRASH
            return 0

        lax.fori_loop(0, STG_CH, prefill, 0)

        def p2_win(wi, curs):
            pltpu.sync_copy(src_hbm.at[pl.ds(trow0 + wi * 8, 8)], srcw)
            pltpu.sync_copy(dst_hbm.at[pl.ds(trow0 + wi * 8, 8)], dstw)

            def p2_chunk(ci, curs):
                new = list(curs)
                for r in range(8):
                    sv = srcw[r, pl.ds(ci * 16, 16)]
                    dv = dstw[r, pl.ds(ci * 16, 16)]
                    bv = dv // BH
                    dloc = dv - bv * BH
                    for b in range(NBUCK):
                        m = bv == b
                        ind = jnp.where(m, 1, 0)
                        pos = new[b] + lax.cumsum(ind) - 1
                        prow = lax.shift_right_logical(pos, 7)
                        pcol = lax.bitwise_and(pos, 127)
                        plsc.store_scatter(stgs, [prow, pcol], sv, mask=m)
                        plsc.store_scatter(stgd, [prow, pcol], dloc, mask=m)
                        new[b] = new[b] + jnp.sum(ind)
                return tuple(new)

            return lax.fori_loop(0, 8, p2_chunk, curs)

        lax.fori_loop(0, WROWS // 8, p2_win, tuple(offs))

        cv = jnp.zeros((16,), jnp.int32)
        for b in range(NBUCK):
            cv = cv + jnp.where(lanes == b, cnts[b], 0)
        histv[0, pl.ds(0, 16)] = cv
        cp1 = pltpu.async_copy(stgs, rs_hbm.at[tile], sem)
        cp2 = pltpu.async_copy(stgd, rd_hbm.at[tile], sem)
        cp3 = pltpu.async_copy(histv, h_hbm.at[pl.ds(tile, 1)], sem)
        cp1.wait()
        cp2.wait()
        cp3.wait()

    return k(src2d, dst2d)


def _sc_agg(table, runs_src, runs_dst, hist, zeros, w):
    zr = ACC_ROWS // NS

    @functools.partial(
        pl.kernel,
        mesh=_MESH,
        out_type=jax.ShapeDtypeStruct((NC, N_OUT, w), jnp.float32),
        scratch_types=[
            pltpu.VMEM((NBF, WIN), jnp.int32),
            pltpu.VMEM((NBF, WIN), jnp.int32),
            pltpu.VMEM((NBF, WIN, w), jnp.float32),
            pltpu.VMEM((NW, 16), jnp.int32),
            pltpu.VMEM_SHARED((ACC_ROWS, w), jnp.float32),
            pltpu.SemaphoreType.DMA,
        ],
        compiler_params=_SC_PARAMS,
    )
    def k(tbl_hbm, rs_hbm, rd_hbm, h_hbm, z_hbm, p_hbm, idxs, idxd, rows,
          histm, accum, sem):
        core = lax.axis_index("c")
        sub = lax.axis_index("s")
        tile = sub * NC + core
        lanes = lax.iota(jnp.int32, 16)

        pltpu.sync_copy(h_hbm, histm)
        hv = histm[tile, pl.ds(0, 16)]

        for b in range(NBUCK):
            pltpu.async_copy(z_hbm, accum.at[pl.ds(sub * zr, zr)], sem).wait()
            plsc.subcore_barrier()

            offch = jnp.int32(0)
            for bp in range(b):
                cb = jnp.sum(jnp.where(lanes == bp, hv, 0))
                offch = offch + _cnt_padded(cb) // WIN
            cnt = jnp.sum(jnp.where(lanes == b, hv, 0))
            ng = _cnt_padded(cnt) // RUN_PAD

            def group(g, _):
                base = offch + g * NBF
                pltpu.sync_copy(rs_hbm.at[tile].at[pl.ds(base, NBF)], idxs)
                pltpu.sync_copy(rd_hbm.at[tile].at[pl.ds(base, NBF)], idxd)
                cps = [
                    pltpu.async_copy(tbl_hbm.at[idxs.at[j]], rows.at[j], sem)
                    for j in range(NBF)
                ]
                for cp in cps:
                    cp.wait()
                for j in range(NBF):
                    pltpu.sync_copy(rows.at[j], accum.at[idxd.at[j]], add=True)
                return 0

            lax.fori_loop(0, ng, group, 0)
            plsc.subcore_barrier()

            dr = BH // NS
            pltpu.async_copy(
                accum.at[pl.ds(sub * dr, dr)],
                p_hbm.at[core].at[pl.ds(b * BH + sub * dr, dr)],
                sem,
            ).wait()
            plsc.subcore_barrier()

    return k(table, runs_src, runs_dst, hist, zeros)


BN = 2000


def _tc_layer1(p1, x, wl, wr, b):
    def body(p_ref, x_ref, wl_ref, wr_ref, b_ref, h_ref, dinv_ref):
        p = p_ref[0] + p_ref[1]
        dinv = 1.0 / jnp.maximum(p[:, 8:9], 1.0)
        agg = p[:, :D_IN] * dinv
        h = (
            jnp.dot(agg, wl_ref[...], preferred_element_type=jnp.float32)
            + jnp.dot(x_ref[...], wr_ref[...], preferred_element_type=jnp.float32)
            + b_ref[...]
        )
        h_ref[...] = jnp.maximum(h, 0.0)
        dinv_ref[...] = dinv

    return pl.pallas_call(
        body,
        grid=(N // BN,),
        in_specs=[
            pl.BlockSpec((NC, BN, 16), lambda i: (0, i, 0)),
            pl.BlockSpec((BN, D_IN), lambda i: (i, 0)),
            pl.BlockSpec((D_IN, D_HID), lambda i: (0, 0)),
            pl.BlockSpec((D_IN, D_HID), lambda i: (0, 0)),
            pl.BlockSpec((1, D_HID), lambda i: (0, 0)),
        ],
        out_specs=[
            pl.BlockSpec((BN, D_HID), lambda i: (i, 0)),
            pl.BlockSpec((BN, 1), lambda i: (i, 0)),
        ],
        out_shape=[
            jax.ShapeDtypeStruct((N, D_HID), jnp.float32),
            jax.ShapeDtypeStruct((N, 1), jnp.float32),
        ],
    )(p1, x, wl, wr, b)


def _tc_layer(p, hprev, dinv, wl, wr, b, relu):
    def body(p_ref, h_ref, d_ref, wl_ref, wr_ref, b_ref, o_ref):
        agg = (p_ref[0] + p_ref[1]) * d_ref[...]
        h = h_ref[...]
        out = (
            jnp.dot(agg, wl_ref[...], preferred_element_type=jnp.float32)
            + jnp.dot(h, wr_ref[...], preferred_element_type=jnp.float32)
            + b_ref[...]
            + h
        )
        o_ref[...] = jnp.maximum(out, 0.0) if relu else out

    return pl.pallas_call(
        body,
        grid=(N // BN,),
        in_specs=[
            pl.BlockSpec((NC, BN, D_HID), lambda i: (0, i, 0)),
            pl.BlockSpec((BN, D_HID), lambda i: (i, 0)),
            pl.BlockSpec((BN, 1), lambda i: (i, 0)),
            pl.BlockSpec((D_HID, D_HID), lambda i: (0, 0)),
            pl.BlockSpec((D_HID, D_HID), lambda i: (0, 0)),
            pl.BlockSpec((1, D_HID), lambda i: (0, 0)),
        ],
        out_specs=[pl.BlockSpec((BN, D_HID), lambda i: (i, 0))],
        out_shape=[jax.ShapeDtypeStruct((N, D_HID), jnp.float32)],
    )(p, hprev, dinv, wl, wr, b)[0]


def kernel(x, edge_index, Wl1, Wr1, b1, Wl2, Wr2, b2, Wl3, Wr3, b3):
    src = edge_index[0].astype(jnp.int32)
    dst = edge_index[1].astype(jnp.int32)
    pad = E_PAD - E
    pad_i = jnp.arange(pad, dtype=jnp.int32)
    src_p = jnp.concatenate([src, (pad_i * 131) % N])
    dst_p = jnp.concatenate([dst, N + pad_i % (NBUCK * BH - N)])
    src2d = src_p.reshape(E_PAD // WIN, WIN)
    dst2d = dst_p.reshape(E_PAD // WIN, WIN)

    xpad = jnp.concatenate(
        [x, jnp.ones((N, 1), jnp.float32), jnp.zeros((N, 7), jnp.float32)], axis=1
    )

    b1r = b1.reshape(1, D_HID)
    b2r = b2.reshape(1, D_HID)
    b3r = b3.reshape(1, D_HID)
    z16 = jnp.zeros((ACC_ROWS // NS, 16), jnp.float32)
    z128 = jnp.zeros((ACC_ROWS // NS, D_HID), jnp.float32)

    runs_s, runs_d, hist = _sc_sort(src2d, dst2d)

    p1 = _sc_agg(xpad, runs_s, runs_d, hist, z16, 16)
    h1, dinv = _tc_layer1(p1, x, Wl1, Wr1, b1r)

    p2 = _sc_agg(h1, runs_s, runs_d, hist, z128, D_HID)
    h2 = _tc_layer(p2, h1, dinv, Wl2, Wr2, b2r, relu=True)

    p3 = _sc_agg(h2, runs_s, runs_d, hist, z128, D_HID)
    return _tc_layer(p3, h2, dinv, Wl3, Wr3, b3r, relu=False)


# speedup vs baseline: 5.7012x; 1.0320x over previous
import dataclasses
import functools

import jax
import jax.numpy as jnp
from jax import lax
from jax.experimental import pallas as pl
from jax.experimental.pallas import tpu as pltpu
from jax.experimental.pallas import tpu_sc as plsc

N = 50000
E = 800000
D_IN = 8
D_HID = 128

NC = 2
NS = 16
NW = NC * NS

WIN = 128
NBF = 4
RUN_PAD = NBF * WIN
EPT = 25600
E_PAD = NW * EPT
WROWS = EPT // WIN

NBUCK = 8
BH = 6400
ACC_TRASH = 128
ACC_ROWS = BH + ACC_TRASH
N_OUT = NBUCK * BH
STG_CH = (EPT + NBUCK * (RUN_PAD - 1) + WIN - 1) // WIN

_MESH = plsc.VectorSubcoreMesh(core_axis_name="c", subcore_axis_name="s")

_SC_PARAMS = pltpu.CompilerParams(use_tc_tiling_on_sc=False)
if "needs_layout_passes" in pltpu.CompilerParams.__dataclass_fields__:
    _SC_PARAMS = dataclasses.replace(_SC_PARAMS, needs_layout_passes=False)


def _cnt_padded(cnt):
    return ((cnt + RUN_PAD - 1) // RUN_PAD) * RUN_PAD


def _sc_sort(src2d, dst2d):
    @functools.partial(
        pl.kernel,
        mesh=_MESH,
        out_type=[
            jax.ShapeDtypeStruct((NW, STG_CH, WIN), jnp.int32),
            jax.ShapeDtypeStruct((NW, STG_CH, WIN), jnp.int32),
            jax.ShapeDtypeStruct((NW, 16), jnp.int32),
        ],
        scratch_types=[
            pltpu.VMEM((8, WIN), jnp.int32),
            pltpu.VMEM((8, WIN), jnp.int32),
            pltpu.VMEM((STG_CH, WIN), jnp.int32),
            pltpu.VMEM((STG_CH, WIN), jnp.int32),
            pltpu.VMEM((1, 16), jnp.int32),
            pltpu.SemaphoreType.DMA,
        ],
        compiler_params=_SC_PARAMS,
    )
    def k(src_hbm, dst_hbm, rs_hbm, rd_hbm, h_hbm, srcw, dstw, stgs, stgd,
          histv, sem):
        core = lax.axis_index("c")
        sub = lax.axis_index("s")
        tile = sub * NC + core
        trow0 = tile * WROWS
        lanes = lax.iota(jnp.int32, 16)

        def p1_win(wi, cnts):
            pltpu.sync_copy(dst_hbm.at[pl.ds(trow0 + wi * 8, 8)], dstw)
            new = list(cnts)
            for r in range(8):
                for ci in range(8):
                    dv = dstw[r, pl.ds(ci * 16, 16)]
                    bv = dv // BH
                    for b in range(NBUCK):
                        ind = jnp.where(bv == b, 1, 0)
                        new[b] = new[b] + jnp.sum(ind)
            return tuple(new)

        cnts = lax.fori_loop(0, WROWS // 8, p1_win,
                             tuple(jnp.int32(0) for _ in range(NBUCK)))

        offs = []
        acc = jnp.int32(0)
        for b in range(NBUCK):
            offs.append(acc)
            acc = acc + _cnt_padded(cnts[b])

        def prefill(i, _):
            for c in range(8):
                base = i * 128 + c * 16
                stgs[i, pl.ds(c * 16, 16)] = (base + lanes * 137) % N
                stgd[i, pl.ds(c * 16, 16)] = BH + (base + lanes) % ACC_TRASH
            return 0

        lax.fori_loop(0, STG_CH, prefill, 0)

        def p2_win(wi, curs):
            pltpu.sync_copy(src_hbm.at[pl.ds(trow0 + wi * 8, 8)], srcw)
            pltpu.sync_copy(dst_hbm.at[pl.ds(trow0 + wi * 8, 8)], dstw)

            def p2_chunk(ci, curs):
                new = list(curs)
                for r in range(8):
                    sv = srcw[r, pl.ds(ci * 16, 16)]
                    dv = dstw[r, pl.ds(ci * 16, 16)]
                    bv = dv // BH
                    dloc = dv - bv * BH
                    for b in range(NBUCK):
                        m = bv == b
                        ind = jnp.where(m, 1, 0)
                        pos = new[b] + lax.cumsum(ind) - 1
                        prow = lax.shift_right_logical(pos, 7)
                        pcol = lax.bitwise_and(pos, 127)
                        plsc.store_scatter(stgs, [prow, pcol], sv, mask=m)
                        plsc.store_scatter(stgd, [prow, pcol], dloc, mask=m)
                        new[b] = new[b] + jnp.sum(ind)
                return tuple(new)

            return lax.fori_loop(0, 8, p2_chunk, curs)

        lax.fori_loop(0, WROWS // 8, p2_win, tuple(offs))

        cv = jnp.zeros((16,), jnp.int32)
        for b in range(NBUCK):
            cv = cv + jnp.where(lanes == b, cnts[b], 0)
        histv[0, pl.ds(0, 16)] = cv
        cp1 = pltpu.async_copy(stgs, rs_hbm.at[tile], sem)
        cp2 = pltpu.async_copy(stgd, rd_hbm.at[tile], sem)
        cp3 = pltpu.async_copy(histv, h_hbm.at[pl.ds(tile, 1)], sem)
        cp1.wait()
        cp2.wait()
        cp3.wait()

    return k(src2d, dst2d)


def _sc_agg(table, runs_src, runs_dst, hist, zeros, w):
    zr = ACC_ROWS // NS
    GP = 2

    @functools.partial(
        pl.kernel,
        mesh=_MESH,
        out_type=jax.ShapeDtypeStruct((NC, N_OUT, w), jnp.float32),
        scratch_types=[
            pltpu.VMEM((2, GP, WIN), jnp.int32),
            pltpu.VMEM((2, GP, WIN), jnp.int32),
            pltpu.VMEM((2, GP, WIN, w), jnp.float32),
            pltpu.VMEM((NW, 16), jnp.int32),
            pltpu.VMEM_SHARED((ACC_ROWS, w), jnp.float32),
            pltpu.SemaphoreType.DMA,
            pltpu.SemaphoreType.DMA,
            pltpu.SemaphoreType.DMA,
        ],
        compiler_params=_SC_PARAMS,
    )
    def k(tbl_hbm, rs_hbm, rd_hbm, h_hbm, z_hbm, p_hbm, idxs, idxd, rows,
          histm, accum, semg, sems0, sems1):
        core = lax.axis_index("c")
        sub = lax.axis_index("s")
        tile = sub * NC + core
        lanes = lax.iota(jnp.int32, 16)
        ssem = (sems0, sems1)

        pltpu.sync_copy(h_hbm, histm)
        hv = histm[tile, pl.ds(0, 16)]

        def drain_scatters(par):
            for j in range(GP):
                pltpu.make_async_copy(
                    rows.at[par].at[j], accum.at[idxd.at[par].at[j]], ssem[par]
                ).wait()

        for b in range(NBUCK):
            pltpu.async_copy(z_hbm, accum.at[pl.ds(sub * zr, zr)], semg).wait()
            plsc.subcore_barrier()

            offch = jnp.int32(0)
            for bp in range(b):
                cb = jnp.sum(jnp.where(lanes == bp, hv, 0))
                offch = offch + _cnt_padded(cb) // WIN
            cnt = jnp.sum(jnp.where(lanes == b, hv, 0))
            npair = _cnt_padded(cnt) // RUN_PAD

            def pair(g2, _):
                for par in range(2):
                    @pl.when(g2 > 0)
                    def _():
                        drain_scatters(par)

                    base = offch + (g2 * 2 + par) * GP
                    pltpu.sync_copy(rs_hbm.at[tile].at[pl.ds(base, GP)],
                                    idxs.at[par])
                    pltpu.sync_copy(rd_hbm.at[tile].at[pl.ds(base, GP)],
                                    idxd.at[par])
                    gcps = [
                        pltpu.async_copy(tbl_hbm.at[idxs.at[par].at[j]],
                                         rows.at[par].at[j], semg)
                        for j in range(GP)
                    ]
                    for cp in gcps:
                        cp.wait()
                    for j in range(GP):
                        pltpu.async_copy(rows.at[par].at[j],
                                         accum.at[idxd.at[par].at[j]],
                                         ssem[par], add=True)
                return 0

            lax.fori_loop(0, npair, pair, 0)

            @pl.when(npair > 0)
            def _():
                drain_scatters(0)
                drain_scatters(1)

            plsc.subcore_barrier()

            dr = BH // NS
            pltpu.async_copy(
                accum.at[pl.ds(sub * dr, dr)],
                p_hbm.at[core].at[pl.ds(b * BH + sub * dr, dr)],
                semg,
            ).wait()
            plsc.subcore_barrier()

    return k(table, runs_src, runs_dst, hist, zeros)


BN = 2000


def _tc_layer1(p1, x, wl, wr, b):
    def body(p_ref, x_ref, wl_ref, wr_ref, b_ref, h_ref, dinv_ref):
        p = p_ref[0] + p_ref[1]
        dinv = 1.0 / jnp.maximum(p[:, 8:9], 1.0)
        agg = p[:, :D_IN] * dinv
        h = (
            jnp.dot(agg, wl_ref[...], preferred_element_type=jnp.float32)
            + jnp.dot(x_ref[...], wr_ref[...], preferred_element_type=jnp.float32)
            + b_ref[...]
        )
        h_ref[...] = jnp.maximum(h, 0.0)
        dinv_ref[...] = dinv

    return pl.pallas_call(
        body,
        grid=(N // BN,),
        in_specs=[
            pl.BlockSpec((NC, BN, 16), lambda i: (0, i, 0)),
            pl.BlockSpec((BN, D_IN), lambda i: (i, 0)),
            pl.BlockSpec((D_IN, D_HID), lambda i: (0, 0)),
            pl.BlockSpec((D_IN, D_HID), lambda i: (0, 0)),
            pl.BlockSpec((1, D_HID), lambda i: (0, 0)),
        ],
        out_specs=[
            pl.BlockSpec((BN, D_HID), lambda i: (i, 0)),
            pl.BlockSpec((BN, 1), lambda i: (i, 0)),
        ],
        out_shape=[
            jax.ShapeDtypeStruct((N, D_HID), jnp.float32),
            jax.ShapeDtypeStruct((N, 1), jnp.float32),
        ],
    )(p1, x, wl, wr, b)


def _tc_layer(p, hprev, dinv, wl, wr, b, relu):
    def body(p_ref, h_ref, d_ref, wl_ref, wr_ref, b_ref, o_ref):
        agg = (p_ref[0] + p_ref[1]) * d_ref[...]
        h = h_ref[...]
        out = (
            jnp.dot(agg, wl_ref[...], preferred_element_type=jnp.float32)
            + jnp.dot(h, wr_ref[...], preferred_element_type=jnp.float32)
            + b_ref[...]
            + h
        )
        o_ref[...] = jnp.maximum(out, 0.0) if relu else out

    return pl.pallas_call(
        body,
        grid=(N // BN,),
        in_specs=[
            pl.BlockSpec((NC, BN, D_HID), lambda i: (0, i, 0)),
            pl.BlockSpec((BN, D_HID), lambda i: (i, 0)),
            pl.BlockSpec((BN, 1), lambda i: (i, 0)),
            pl.BlockSpec((D_HID, D_HID), lambda i: (0, 0)),
            pl.BlockSpec((D_HID, D_HID), lambda i: (0, 0)),
            pl.BlockSpec((1, D_HID), lambda i: (0, 0)),
        ],
        out_specs=[pl.BlockSpec((BN, D_HID), lambda i: (i, 0))],
        out_shape=[jax.ShapeDtypeStruct((N, D_HID), jnp.float32)],
    )(p, hprev, dinv, wl, wr, b)[0]


def kernel(x, edge_index, Wl1, Wr1, b1, Wl2, Wr2, b2, Wl3, Wr3, b3):
    src = edge_index[0].astype(jnp.int32)
    dst = edge_index[1].astype(jnp.int32)
    pad = E_PAD - E
    pad_i = jnp.arange(pad, dtype=jnp.int32)
    src_p = jnp.concatenate([src, (pad_i * 131) % N])
    dst_p = jnp.concatenate([dst, N + pad_i % (NBUCK * BH - N)])
    src2d = src_p.reshape(E_PAD // WIN, WIN)
    dst2d = dst_p.reshape(E_PAD // WIN, WIN)

    xpad = jnp.concatenate(
        [x, jnp.ones((N, 1), jnp.float32), jnp.zeros((N, 7), jnp.float32)], axis=1
    )

    b1r = b1.reshape(1, D_HID)
    b2r = b2.reshape(1, D_HID)
    b3r = b3.reshape(1, D_HID)
    z16 = jnp.zeros((ACC_ROWS // NS, 16), jnp.float32)
    z128 = jnp.zeros((ACC_ROWS // NS, D_HID), jnp.float32)

    runs_s, runs_d, hist = _sc_sort(src2d, dst2d)

    p1 = _sc_agg(xpad, runs_s, runs_d, hist, z16, 16)
    h1, dinv = _tc_layer1(p1, x, Wl1, Wr1, b1r)

    p2 = _sc_agg(h1, runs_s, runs_d, hist, z128, D_HID)
    h2 = _tc_layer(p2, h1, dinv, Wl2, Wr2, b2r, relu=True)

    p3 = _sc_agg(h2, runs_s, runs_d, hist, z128, D_HID)
    return _tc_layer(p3, h2, dinv, Wl3, Wr3, b3r, relu=False)


# speedup vs baseline: 6.5334x; 1.1460x over previous
import dataclasses
import functools

import jax
import jax.numpy as jnp
from jax import lax
from jax.experimental import pallas as pl
from jax.experimental.pallas import tpu as pltpu
from jax.experimental.pallas import tpu_sc as plsc

N = 50000
E = 800000
D_IN = 8
D_HID = 128

NC = 2
NS = 16
NW = NC * NS

WIN = 128
NBF = 4
RUN_PAD = NBF * WIN
EPT = 25600
E_PAD = NW * EPT
WROWS = EPT // WIN

NBUCK = 8
BH = 6400
ACC_TRASH = 128
ACC_ROWS = BH + ACC_TRASH
N_OUT = NBUCK * BH
CW = 256
STG_CH = (EPT + NBUCK * (RUN_PAD - 1) + CW - 1) // CW

_MESH = plsc.VectorSubcoreMesh(core_axis_name="c", subcore_axis_name="s")

_SC_PARAMS = pltpu.CompilerParams(use_tc_tiling_on_sc=False)
if "needs_layout_passes" in pltpu.CompilerParams.__dataclass_fields__:
    _SC_PARAMS = dataclasses.replace(_SC_PARAMS, needs_layout_passes=False)


def _cnt_padded(cnt):
    return ((cnt + RUN_PAD - 1) // RUN_PAD) * RUN_PAD


def _sc_sort(src2d, dst2d):
    @functools.partial(
        pl.kernel,
        mesh=_MESH,
        out_type=[
            jax.ShapeDtypeStruct((NW, STG_CH, CW), jnp.int32),
            jax.ShapeDtypeStruct((NW, STG_CH, CW), jnp.int32),
            jax.ShapeDtypeStruct((NW, 16), jnp.int32),
        ],
        scratch_types=[
            pltpu.VMEM((8, WIN), jnp.int32),
            pltpu.VMEM((8, WIN), jnp.int32),
            pltpu.VMEM((STG_CH, CW), jnp.int32),
            pltpu.VMEM((STG_CH, CW), jnp.int32),
            pltpu.VMEM((1, 16), jnp.int32),
            pltpu.SemaphoreType.DMA,
        ],
        compiler_params=_SC_PARAMS,
    )
    def k(src_hbm, dst_hbm, rs_hbm, rd_hbm, h_hbm, srcw, dstw, stgs, stgd,
          histv, sem):
        core = lax.axis_index("c")
        sub = lax.axis_index("s")
        tile = sub * NC + core
        trow0 = tile * WROWS
        lanes = lax.iota(jnp.int32, 16)

        def p1_win(wi, cnts):
            pltpu.sync_copy(dst_hbm.at[pl.ds(trow0 + wi * 8, 8)], dstw)
            new = list(cnts)
            for r in range(8):
                for ci in range(8):
                    dv = dstw[r, pl.ds(ci * 16, 16)]
                    bv = dv // BH
                    for b in range(NBUCK):
                        ind = jnp.where(bv == b, 1, 0)
                        new[b] = new[b] + jnp.sum(ind)
            return tuple(new)

        cnts = lax.fori_loop(0, WROWS // 8, p1_win,
                             tuple(jnp.int32(0) for _ in range(NBUCK)))

        offs = []
        acc = jnp.int32(0)
        for b in range(NBUCK):
            offs.append(acc)
            acc = acc + _cnt_padded(cnts[b])

        def prefill(i, _):
            for c in range(16):
                base = i * CW + c * 16
                stgs[i, pl.ds(c * 16, 16)] = (base + lanes * 137) % N
                stgd[i, pl.ds(c * 16, 16)] = BH + (base + lanes) % ACC_TRASH
            return 0

        lax.fori_loop(0, STG_CH, prefill, 0)

        def p2_win(wi, curs):
            pltpu.sync_copy(src_hbm.at[pl.ds(trow0 + wi * 8, 8)], srcw)
            pltpu.sync_copy(dst_hbm.at[pl.ds(trow0 + wi * 8, 8)], dstw)

            def p2_chunk(ci, curs):
                new = list(curs)
                for r in range(8):
                    sv = srcw[r, pl.ds(ci * 16, 16)]
                    dv = dstw[r, pl.ds(ci * 16, 16)]
                    bv = dv // BH
                    dloc = dv - bv * BH
                    for b in range(NBUCK):
                        m = bv == b
                        ind = jnp.where(m, 1, 0)
                        pos = new[b] + lax.cumsum(ind) - 1
                        prow = lax.shift_right_logical(pos, 8)
                        pcol = lax.bitwise_and(pos, 255)
                        plsc.store_scatter(stgs, [prow, pcol], sv, mask=m)
                        plsc.store_scatter(stgd, [prow, pcol], dloc, mask=m)
                        new[b] = new[b] + jnp.sum(ind)
                return tuple(new)

            return lax.fori_loop(0, 8, p2_chunk, curs)

        lax.fori_loop(0, WROWS // 8, p2_win, tuple(offs))

        cv = jnp.zeros((16,), jnp.int32)
        for b in range(NBUCK):
            cv = cv + jnp.where(lanes == b, cnts[b], 0)
        histv[0, pl.ds(0, 16)] = cv
        cp1 = pltpu.async_copy(stgs, rs_hbm.at[tile], sem)
        cp2 = pltpu.async_copy(stgd, rd_hbm.at[tile], sem)
        cp3 = pltpu.async_copy(histv, h_hbm.at[pl.ds(tile, 1)], sem)
        cp1.wait()
        cp2.wait()
        cp3.wait()

    return k(src2d, dst2d)


def _sc_agg(table, runs_src, runs_dst, hist, zeros, w):
    zr = ACC_ROWS // NS

    @functools.partial(
        pl.kernel,
        mesh=_MESH,
        out_type=jax.ShapeDtypeStruct((NC, N_OUT, w), jnp.float32),
        scratch_types=[
            pltpu.VMEM((2, 1, CW), jnp.int32),
            pltpu.VMEM((2, 1, CW), jnp.int32),
            pltpu.VMEM((2, CW, w), jnp.float32),
            pltpu.VMEM((NW, 16), jnp.int32),
            pltpu.VMEM_SHARED((ACC_ROWS, w), jnp.float32),
            pltpu.SemaphoreType.DMA,
            pltpu.SemaphoreType.DMA,
            pltpu.SemaphoreType.DMA,
        ],
        compiler_params=_SC_PARAMS,
    )
    def k(tbl_hbm, rs_hbm, rd_hbm, h_hbm, z_hbm, p_hbm, idxs, idxd, rows,
          histm, accum, semg, sems0, sems1):
        core = lax.axis_index("c")
        sub = lax.axis_index("s")
        tile = sub * NC + core
        lanes = lax.iota(jnp.int32, 16)
        ssem = (sems0, sems1)

        pltpu.sync_copy(h_hbm, histm)
        hv = histm[tile, pl.ds(0, 16)]

        def drain_scatters(par):
            pltpu.make_async_copy(
                rows.at[par], accum.at[idxd.at[par].at[0]], ssem[par]
            ).wait()

        for b in range(NBUCK):
            pltpu.async_copy(z_hbm, accum.at[pl.ds(sub * zr, zr)], semg).wait()
            plsc.subcore_barrier()

            offch = jnp.int32(0)
            for bp in range(b):
                cb = jnp.sum(jnp.where(lanes == bp, hv, 0))
                offch = offch + _cnt_padded(cb) // CW
            cnt = jnp.sum(jnp.where(lanes == b, hv, 0))
            npair = _cnt_padded(cnt) // RUN_PAD

            def pair(g2, _):
                gcps = []
                for par in range(2):
                    @pl.when(g2 > 0)
                    def _():
                        drain_scatters(par)

                    base = offch + g2 * 2 + par
                    pltpu.sync_copy(rs_hbm.at[tile].at[pl.ds(base, 1)],
                                    idxs.at[par])
                    pltpu.sync_copy(rd_hbm.at[tile].at[pl.ds(base, 1)],
                                    idxd.at[par])
                    gcps.append(
                        pltpu.async_copy(tbl_hbm.at[idxs.at[par].at[0]],
                                         rows.at[par], semg)
                    )
                for par in range(2):
                    gcps[par].wait()
                    pltpu.async_copy(rows.at[par], accum.at[idxd.at[par].at[0]],
                                     ssem[par], add=True)
                return 0

            lax.fori_loop(0, npair, pair, 0)

            @pl.when(npair > 0)
            def _():
                drain_scatters(0)
                drain_scatters(1)

            plsc.subcore_barrier()

            dr = BH // NS
            pltpu.async_copy(
                accum.at[pl.ds(sub * dr, dr)],
                p_hbm.at[core].at[pl.ds(b * BH + sub * dr, dr)],
                semg,
            ).wait()
            plsc.subcore_barrier()

    return k(table, runs_src, runs_dst, hist, zeros)


BN = 2000


def _tc_layer1(p1, x, wl, wr, b):
    def body(p_ref, x_ref, wl_ref, wr_ref, b_ref, h_ref, dinv_ref):
        p = p_ref[0] + p_ref[1]
        dinv = 1.0 / jnp.maximum(p[:, 8:9], 1.0)
        agg = p[:, :D_IN] * dinv
        h = (
            jnp.dot(agg, wl_ref[...], preferred_element_type=jnp.float32)
            + jnp.dot(x_ref[...], wr_ref[...], preferred_element_type=jnp.float32)
            + b_ref[...]
        )
        h_ref[...] = jnp.maximum(h, 0.0)
        dinv_ref[...] = dinv

    return pl.pallas_call(
        body,
        grid=(N // BN,),
        in_specs=[
            pl.BlockSpec((NC, BN, 16), lambda i: (0, i, 0)),
            pl.BlockSpec((BN, D_IN), lambda i: (i, 0)),
            pl.BlockSpec((D_IN, D_HID), lambda i: (0, 0)),
            pl.BlockSpec((D_IN, D_HID), lambda i: (0, 0)),
            pl.BlockSpec((1, D_HID), lambda i: (0, 0)),
        ],
        out_specs=[
            pl.BlockSpec((BN, D_HID), lambda i: (i, 0)),
            pl.BlockSpec((BN, 1), lambda i: (i, 0)),
        ],
        out_shape=[
            jax.ShapeDtypeStruct((N, D_HID), jnp.float32),
            jax.ShapeDtypeStruct((N, 1), jnp.float32),
        ],
    )(p1, x, wl, wr, b)


def _tc_layer(p, hprev, dinv, wl, wr, b, relu):
    def body(p_ref, h_ref, d_ref, wl_ref, wr_ref, b_ref, o_ref):
        agg = (p_ref[0] + p_ref[1]) * d_ref[...]
        h = h_ref[...]
        out = (
            jnp.dot(agg, wl_ref[...], preferred_element_type=jnp.float32)
            + jnp.dot(h, wr_ref[...], preferred_element_type=jnp.float32)
            + b_ref[...]
            + h
        )
        o_ref[...] = jnp.maximum(out, 0.0) if relu else out

    return pl.pallas_call(
        body,
        grid=(N // BN,),
        in_specs=[
            pl.BlockSpec((NC, BN, D_HID), lambda i: (0, i, 0)),
            pl.BlockSpec((BN, D_HID), lambda i: (i, 0)),
            pl.BlockSpec((BN, 1), lambda i: (i, 0)),
            pl.BlockSpec((D_HID, D_HID), lambda i: (0, 0)),
            pl.BlockSpec((D_HID, D_HID), lambda i: (0, 0)),
            pl.BlockSpec((1, D_HID), lambda i: (0, 0)),
        ],
        out_specs=[pl.BlockSpec((BN, D_HID), lambda i: (i, 0))],
        out_shape=[jax.ShapeDtypeStruct((N, D_HID), jnp.float32)],
    )(p, hprev, dinv, wl, wr, b)[0]


def kernel(x, edge_index, Wl1, Wr1, b1, Wl2, Wr2, b2, Wl3, Wr3, b3):
    src = edge_index[0].astype(jnp.int32)
    dst = edge_index[1].astype(jnp.int32)
    pad = E_PAD - E
    pad_i = jnp.arange(pad, dtype=jnp.int32)
    src_p = jnp.concatenate([src, (pad_i * 131) % N])
    dst_p = jnp.concatenate([dst, N + pad_i % (NBUCK * BH - N)])
    src2d = src_p.reshape(E_PAD // WIN, WIN)
    dst2d = dst_p.reshape(E_PAD // WIN, WIN)

    xpad = jnp.concatenate(
        [x, jnp.ones((N, 1), jnp.float32), jnp.zeros((N, 7), jnp.float32)], axis=1
    )

    b1r = b1.reshape(1, D_HID)
    b2r = b2.reshape(1, D_HID)
    b3r = b3.reshape(1, D_HID)
    z16 = jnp.zeros((ACC_ROWS // NS, 16), jnp.float32)
    z128 = jnp.zeros((ACC_ROWS // NS, D_HID), jnp.float32)

    runs_s, runs_d, hist = _sc_sort(src2d, dst2d)

    p1 = _sc_agg(xpad, runs_s, runs_d, hist, z16, 16)
    h1, dinv = _tc_layer1(p1, x, Wl1, Wr1, b1r)

    p2 = _sc_agg(h1, runs_s, runs_d, hist, z128, D_HID)
    h2 = _tc_layer(p2, h1, dinv, Wl2, Wr2, b2r, relu=True)

    p3 = _sc_agg(h2, runs_s, runs_d, hist, z128, D_HID)
    return _tc_layer(p3, h2, dinv, Wl3, Wr3, b3r, relu=False)
